# baseline (speedup 1.0000x reference)
import numpy as np

import jax
import jax.numpy as jnp
from jax.experimental import pallas as pl
from jax.experimental.pallas import tpu as pltpu
from jax.sharding import Mesh, NamedSharding, PartitionSpec as P

_TEMP = 0.05
_NB = 5120


def _logits_t_body(xt_ref, ft_ref, o_ref):
    xt = xt_ref[...]
    norm = jnp.sqrt(jnp.sum(xt * xt, axis=0, keepdims=True))
    xs = xt * ((1.0 / _TEMP) / jnp.maximum(norm, 1e-12))
    o_ref[...] = jax.lax.dot_general(
        ft_ref[...],
        xs,
        (((0,), (0,)), ((), ())),
        preferred_element_type=jnp.float32,
    )


def _logits_t_shard(xt, ft):
    d, b = xt.shape
    n = ft.shape[1]
    return pl.pallas_call(
        _logits_t_body,
        grid=(pl.cdiv(n, _NB),),
        in_specs=[
            pl.BlockSpec((d, b), lambda i: (0, 0)),
            pl.BlockSpec((d, _NB), lambda i: (0, i)),
        ],
        out_specs=pl.BlockSpec((_NB, b), lambda i: (i, 0)),
        out_shape=jax.ShapeDtypeStruct((n, b), jnp.float32),
        compiler_params=pltpu.CompilerParams(
            dimension_semantics=("arbitrary",),
        ),
    )(xt, ft)


def kernel(inputs, targets, features):
    del targets
    xt = jnp.transpose(inputs)
    ft = jnp.transpose(features)
    devs = jax.devices()
    if len(devs) > 1:
        mesh = Mesh(np.asarray(devs[:2]), ("x",))
        out_t = jax.shard_map(
            _logits_t_shard,
            mesh=mesh,
            in_specs=(P(None, None), P(None, "x")),
            out_specs=P("x", None),
            check_vma=False,
        )(xt, ft)
    else:
        out_t = _logits_t_shard(xt, ft)
    return jnp.transpose(out_t)


# speedup vs baseline: 3.0338x; 3.0338x over previous
import jax
import jax.numpy as jnp
from jax.experimental import pallas as pl
from jax.experimental.pallas import tpu as pltpu

_TEMP = 0.05
_NB = 6400


def _logits_t_body(xt_ref, ft_ref, o_ref):
    xt = xt_ref[...]
    norm = jnp.sqrt(jnp.sum(xt * xt, axis=0, keepdims=True))
    xs = xt * ((1.0 / _TEMP) / jnp.maximum(norm, 1e-12))
    o_ref[...] = jax.lax.dot_general(
        ft_ref[...],
        xs,
        (((0,), (0,)), ((), ())),
        preferred_element_type=jnp.float32,
    )


def _logits_t(xt, ft):
    d, b = xt.shape
    n = ft.shape[1]
    return pl.pallas_call(
        _logits_t_body,
        grid=(pl.cdiv(n, _NB),),
        in_specs=[
            pl.BlockSpec((d, b), lambda i: (0, 0)),
            pl.BlockSpec((d, _NB), lambda i: (0, i)),
        ],
        out_specs=pl.BlockSpec((_NB, b), lambda i: (i, 0)),
        out_shape=jax.ShapeDtypeStruct((n, b), jnp.float32),
        compiler_params=pltpu.CompilerParams(
            dimension_semantics=("arbitrary",),
        ),
    )(xt, ft)


def kernel(inputs, targets, features):
    del targets
    xt = jnp.transpose(inputs)
    ft = jnp.transpose(features)
    out_t = _logits_t(xt, ft)
    return jnp.transpose(out_t)


# speedup vs baseline: 3.0493x; 1.0051x over previous
import jax
import jax.numpy as jnp
from jax.experimental import pallas as pl
from jax.experimental.pallas import tpu as pltpu

_TEMP = 0.05
_NB = 4096


def _logits_t_body(xt_ref, ft_ref, o_ref):
    xt = xt_ref[...]
    norm = jnp.sqrt(jnp.sum(xt * xt, axis=0, keepdims=True))
    xs = xt * ((1.0 / _TEMP) / jnp.maximum(norm, 1e-12))
    o_ref[...] = jax.lax.dot_general(
        ft_ref[...],
        xs,
        (((0,), (0,)), ((), ())),
        preferred_element_type=jnp.float32,
    )


_K = 3


def _logits_t_body_manual(xt_ref, ft_ref, o_ref, vbuf, sem):
    n = o_ref.shape[0]
    nsteps = (n + _NB - 1) // _NB
    tail = n - (nsteps - 1) * _NB
    i = pl.program_id(0)
    slot = jax.lax.rem(i, _K)

    @pl.when(i >= _K)
    def _():
        prev = i - _K
        pltpu.make_async_copy(
            vbuf.at[jax.lax.rem(prev, _K)],
            o_ref.at[pl.ds(prev * _NB, _NB)],
            sem.at[jax.lax.rem(prev, _K)],
        ).wait()

    xt = xt_ref[...]
    norm = jnp.sqrt(jnp.sum(xt * xt, axis=0, keepdims=True))
    xs = xt * ((1.0 / _TEMP) / jnp.maximum(norm, 1e-12))
    vbuf[slot] = jax.lax.dot_general(
        ft_ref[...],
        xs,
        (((0,), (0,)), ((), ())),
        preferred_element_type=jnp.float32,
    )

    @pl.when(i < nsteps - 1)
    def _():
        pltpu.make_async_copy(
            vbuf.at[slot], o_ref.at[pl.ds(i * _NB, _NB)], sem.at[slot]
        ).start()

    @pl.when(i == nsteps - 1)
    def _():
        last_slot = (nsteps - 1) % _K
        pltpu.make_async_copy(
            vbuf.at[last_slot, pl.ds(0, tail)],
            o_ref.at[pl.ds((nsteps - 1) * _NB, tail)],
            sem.at[last_slot],
        ).start()
        for j in range(max(0, nsteps - _K), nsteps):
            if j == nsteps - 1:
                pltpu.make_async_copy(
                    vbuf.at[j % _K, pl.ds(0, tail)],
                    o_ref.at[pl.ds(j * _NB, tail)],
                    sem.at[j % _K],
                ).wait()
            else:
                pltpu.make_async_copy(
                    vbuf.at[j % _K],
                    o_ref.at[pl.ds(j * _NB, _NB)],
                    sem.at[j % _K],
                ).wait()


def _logits_t(xt, ft):
    d, b = xt.shape
    n = ft.shape[1]
    return pl.pallas_call(
        _logits_t_body_manual,
        grid=(pl.cdiv(n, _NB),),
        in_specs=[
            pl.BlockSpec((d, b), lambda i: (0, 0)),
            pl.BlockSpec((d, _NB), lambda i: (0, i)),
        ],
        out_specs=pl.BlockSpec(memory_space=pl.ANY),
        out_shape=jax.ShapeDtypeStruct((n, b), jnp.float32),
        scratch_shapes=[
            pltpu.VMEM((_K, _NB, b), jnp.float32),
            pltpu.SemaphoreType.DMA((_K,)),
        ],
        compiler_params=pltpu.CompilerParams(
            dimension_semantics=("arbitrary",),
        ),
    )(xt, ft)


def kernel(inputs, targets, features):
    del targets
    xt = jnp.transpose(inputs)
    ft = jnp.transpose(features)
    out_t = _logits_t(xt, ft)
    return jnp.transpose(out_t)


# speedup vs baseline: 3.0495x; 1.0001x over previous
import jax
import jax.numpy as jnp
from jax.experimental import pallas as pl
from jax.experimental.pallas import tpu as pltpu

_TEMP = 0.05
_NB = 4096


def _logits_t_body(xt_ref, ft_ref, o_ref):
    xt = xt_ref[...]
    norm = jnp.sqrt(jnp.sum(xt * xt, axis=0, keepdims=True))
    xs = xt * ((1.0 / _TEMP) / jnp.maximum(norm, 1e-12))
    o_ref[...] = jax.lax.dot_general(
        ft_ref[...],
        xs,
        (((0,), (0,)), ((), ())),
        preferred_element_type=jnp.float32,
    )


_K = 3


def _logits_t_body_manual(xt_ref, ft_ref, o_ref, vbuf, sem):
    n = o_ref.shape[0]
    nsteps = (n + _NB - 1) // _NB
    tail = n - (nsteps - 1) * _NB
    i = pl.program_id(0)
    slot = jax.lax.rem(i, _K)

    for k in range(_K):
        @pl.when(jnp.logical_and(i >= _K, slot == k))
        def _(k=k):
            prev = i - _K
            pltpu.make_async_copy(
                vbuf.at[k],
                o_ref.at[pl.ds(prev * _NB, _NB)],
                sem.at[k],
            ).wait()

    xt = xt_ref[...]
    norm = jnp.sqrt(jnp.sum(xt * xt, axis=0, keepdims=True))
    xs = xt * ((1.0 / _TEMP) / jnp.maximum(norm, 1e-12))
    vbuf[slot] = jax.lax.dot_general(
        ft_ref[...],
        xs,
        (((0,), (0,)), ((), ())),
        preferred_element_type=jnp.float32,
    )

    for k in range(_K):
        @pl.when(jnp.logical_and(i < nsteps - 1, slot == k))
        def _(k=k):
            pltpu.make_async_copy(
                vbuf.at[k], o_ref.at[pl.ds(i * _NB, _NB)], sem.at[k]
            ).start()

    @pl.when(i == nsteps - 1)
    def _():
        last_slot = (nsteps - 1) % _K
        pltpu.make_async_copy(
            vbuf.at[last_slot, pl.ds(0, tail)],
            o_ref.at[pl.ds((nsteps - 1) * _NB, tail)],
            sem.at[last_slot],
        ).start()
        for j in range(max(0, nsteps - _K), nsteps):
            if j == nsteps - 1:
                pltpu.make_async_copy(
                    vbuf.at[j % _K, pl.ds(0, tail)],
                    o_ref.at[pl.ds(j * _NB, tail)],
                    sem.at[j % _K],
                ).wait()
            else:
                pltpu.make_async_copy(
                    vbuf.at[j % _K],
                    o_ref.at[pl.ds(j * _NB, _NB)],
                    sem.at[j % _K],
                ).wait()


def _logits_t(xt, ft):
    d, b = xt.shape
    n = ft.shape[1]
    return pl.pallas_call(
        _logits_t_body_manual,
        grid=(pl.cdiv(n, _NB),),
        in_specs=[
            pl.BlockSpec((d, b), lambda i: (0, 0)),
            pl.BlockSpec((d, _NB), lambda i: (0, i)),
        ],
        out_specs=pl.BlockSpec(memory_space=pl.ANY),
        out_shape=jax.ShapeDtypeStruct((n, b), jnp.float32),
        scratch_shapes=[
            pltpu.VMEM((_K, _NB, b), jnp.float32),
            pltpu.SemaphoreType.DMA((_K,)),
        ],
        compiler_params=pltpu.CompilerParams(
            dimension_semantics=("arbitrary",),
        ),
    )(xt, ft)


def kernel(inputs, targets, features):
    del targets
    xt = jnp.transpose(inputs)
    ft = jnp.transpose(features)
    out_t = _logits_t(xt, ft)
    return jnp.transpose(out_t)


# speedup vs baseline: 3.0560x; 1.0021x over previous
import jax
import jax.numpy as jnp
from jax.experimental import pallas as pl
from jax.experimental.pallas import tpu as pltpu

_TEMP = 0.05
_NB = 4096


def _logits_t_body(xt_ref, ft_ref, o_ref):
    xt = xt_ref[...]
    norm = jnp.sqrt(jnp.sum(xt * xt, axis=0, keepdims=True))
    xs = xt * ((1.0 / _TEMP) / jnp.maximum(norm, 1e-12))
    o_ref[...] = jax.lax.dot_general(
        ft_ref[...],
        xs,
        (((0,), (0,)), ((), ())),
        preferred_element_type=jnp.float32,
    )


def _logits_t(xt, ft):
    d, b = xt.shape
    n = ft.shape[1]
    return pl.pallas_call(
        _logits_t_body,
        grid=(pl.cdiv(n, _NB),),
        in_specs=[
            pl.BlockSpec((d, b), lambda i: (0, 0)),
            pl.BlockSpec((d, _NB), lambda i: (0, i)),
        ],
        out_specs=pl.BlockSpec((_NB, b), lambda i: (i, 0)),
        out_shape=jax.ShapeDtypeStruct((n, b), jnp.float32),
        compiler_params=pltpu.CompilerParams(
            dimension_semantics=("arbitrary",),
        ),
    )(xt, ft)


def kernel(inputs, targets, features):
    del targets
    xt = jnp.transpose(inputs)
    ft = jnp.transpose(features)
    out_t = _logits_t(xt, ft)
    return jnp.transpose(out_t)
